# baseline (speedup 1.0000x reference)
import functools

import jax
import jax.numpy as jnp
from jax import lax
from jax.experimental import pallas as pl
from jax.experimental.pallas import tpu as pltpu
from jax.experimental.pallas import tpu_sc as plsc

R = 128
C = 32768
BC = 4096
NB = C // BC

_BIG = 2**30


def _tc_body(x_ref, z_ref, idx_ref, m_ref, i_ref):
    b = pl.program_id(0)
    z_ref[...] = jnp.zeros_like(z_ref)

    x = x_ref[...]
    bm = jnp.max(x, axis=1, keepdims=True)
    col = lax.broadcasted_iota(jnp.int32, x.shape, 1) + b * BC
    bi = jnp.min(jnp.where(x == bm, col, _BIG), axis=1, keepdims=True)

    @pl.when(b == 0)
    def _():
        m_ref[...] = bm
        i_ref[...] = bi

    @pl.when(b != 0)
    def _():
        better = bm > m_ref[...]
        m_ref[...] = jnp.where(better, bm, m_ref[...])
        i_ref[...] = jnp.where(better, bi, i_ref[...])

    @pl.when(b == NB - 1)
    def _():
        row = lax.broadcasted_iota(jnp.int32, (R, 1), 0)
        idx_ref[...] = i_ref[...] + row * C


def _tc_argmax_zero(x):
    return pl.pallas_call(
        _tc_body,
        grid=(NB,),
        in_specs=[pl.BlockSpec((R, BC), lambda b: (0, b))],
        out_specs=[
            pl.BlockSpec((R, BC), lambda b: (0, b)),
            pl.BlockSpec((R, 1), lambda b: (0, 0)),
        ],
        out_shape=[
            jax.ShapeDtypeStruct((R, C), jnp.float32),
            jax.ShapeDtypeStruct((R, 1), jnp.int32),
        ],
        scratch_shapes=[
            pltpu.VMEM((R, 1), jnp.float32),
            pltpu.VMEM((R, 1), jnp.int32),
        ],
    )(x)


@functools.partial(
    pl.kernel,
    mesh=plsc.VectorSubcoreMesh(core_axis_name="c", subcore_axis_name="s"),
    scratch_types=[
        pltpu.VMEM((R,), jnp.int32),
        pltpu.VMEM((R,), jnp.float32),
        pltpu.SemaphoreType.DMA,
    ],
)
def _sc_scatter(flat_idx_hbm, out_hbm, idx_v, ones_v, sem):
    wid = lax.axis_index("s") * 2 + lax.axis_index("c")

    @pl.when(wid == 0)
    def _():
        pltpu.sync_copy(flat_idx_hbm, idx_v)
        for k in range(R // 16):
            ones_v[pl.ds(16 * k, 16)] = jnp.ones((16,), jnp.float32)
        pltpu.async_copy(ones_v, out_hbm.at[idx_v], sem).wait()


def kernel(input):
    zeros, idx = _tc_argmax_zero(input)
    ref = jax.new_ref(zeros.reshape(R * C))
    _sc_scatter(idx.reshape(R), ref)
    return jax.freeze(ref).reshape(R, C)


# speedup vs baseline: 4.9883x; 4.9883x over previous
import jax
import jax.numpy as jnp
from jax import lax
from jax.experimental import pallas as pl
from jax.experimental.pallas import tpu as pltpu

R = 128
C = 32768
BC = 8192
NB = C // BC

_BIG = 2**30


def _body(x_ref, o_ref, m_ref, i_ref):
    p = pl.program_id(0)
    b = pl.program_id(1)

    @pl.when(p == 0)
    def _pass0():
        x = x_ref[...]
        bm = jnp.max(x, axis=1, keepdims=True)
        col = lax.broadcasted_iota(jnp.int32, x.shape, 1) + b * BC
        bi = jnp.min(jnp.where(x == bm, col, _BIG), axis=1, keepdims=True)

        @pl.when(b == 0)
        def _():
            m_ref[...] = bm
            i_ref[...] = bi

        @pl.when(b != 0)
        def _():
            better = bm > m_ref[...]
            m_ref[...] = jnp.where(better, bm, m_ref[...])
            i_ref[...] = jnp.where(better, bi, i_ref[...])

    @pl.when(p == 1)
    def _pass1():
        col = lax.broadcasted_iota(jnp.int32, o_ref.shape, 1) + b * BC
        o_ref[...] = (col == i_ref[...]).astype(jnp.float32)


def kernel(input):
    return pl.pallas_call(
        _body,
        grid=(2, NB),
        in_specs=[
            pl.BlockSpec((R, BC), lambda p, b: (0, jnp.where(p == 0, b, NB - 1))),
        ],
        out_specs=pl.BlockSpec((R, BC), lambda p, b: (0, jnp.where(p == 0, 0, b))),
        out_shape=jax.ShapeDtypeStruct((R, C), jnp.float32),
        scratch_shapes=[
            pltpu.VMEM((R, 1), jnp.float32),
            pltpu.VMEM((R, 1), jnp.int32),
        ],
    )(input)
